# baseline (speedup 1.0000x reference)
import functools

import jax
import jax.numpy as jnp
from jax import lax
from jax.experimental import pallas as pl
from jax.experimental.pallas import tpu as pltpu
from jax.experimental.pallas import tpu_sc as plsc

IDX_CHUNK = 64


def _chunk_sizes(total: int):
    sizes = []
    for s in (16, 16, 32, 64):
        if sum(sizes) + s <= total:
            sizes.append(s)
    while total - sum(sizes) >= 128:
        sizes.append(128)
    if total - sum(sizes):
        sizes.append(total - sum(sizes))
    return sizes


def _build_gather(batch: int, vocab: int, d: int):
    info = plsc.get_sparse_core_info()
    nw = info.num_cores * info.num_subcores
    b_per_w = batch // nw
    n_chunks = len(_chunk_sizes(b_per_w))
    mesh = plsc.VectorSubcoreMesh(core_axis_name="c", subcore_axis_name="s")

    @functools.partial(
        pl.kernel,
        mesh=mesh,
        out_type=jax.ShapeDtypeStruct((batch, d), jnp.float32),
        scratch_types=[
            pltpu.VMEM((b_per_w,), jnp.int32),
            pltpu.VMEM((b_per_w, d), jnp.float32),
            pltpu.VMEM_SHARED((vocab, d), jnp.float32),
        ]
        + [pltpu.SemaphoreType.DMA] * (n_chunks + 1),
    )
    def gather_kernel(idx_hbm, table_hbm, out_hbm, idx_v, rows_v, table_sh, *sems):
        gsems, ssem = sems[:n_chunks], sems[n_chunks]
        cid = lax.axis_index("c")
        sid = lax.axis_index("s")
        wid = sid * info.num_cores + cid
        base = wid * b_per_w

        half = ((vocab + 15) // 16) * 8
        @pl.when(sid == 0)
        def _():
            pltpu.sync_copy(table_hbm.at[pl.ds(0, half)], table_sh.at[pl.ds(0, half)])

        @pl.when(sid == info.num_subcores // 2)
        def _():
            pltpu.sync_copy(
                table_hbm.at[pl.ds(half, vocab - half)],
                table_sh.at[pl.ds(half, vocab - half)],
            )

        pltpu.sync_copy(idx_hbm.at[pl.ds(base, b_per_w)], idx_v)
        plsc.subcore_barrier()

        sizes = _chunk_sizes(b_per_w)
        offs = [sum(sizes[:j]) for j in range(len(sizes))]
        copies = []
        for j, (o, sz) in enumerate(zip(offs, sizes)):
            copies.append(
                pltpu.async_copy(
                    table_sh.at[idx_v.at[pl.ds(o, sz)]],
                    rows_v.at[pl.ds(o, sz)],
                    gsems[j],
                )
            )
        stores = []
        for j, (o, sz) in enumerate(zip(offs, sizes)):
            copies[j].wait()
            stores.append(
                pltpu.async_copy(
                    rows_v.at[pl.ds(o, sz)],
                    out_hbm.at[pl.ds(base + o, sz)],
                    ssem,
                )
            )
        for s in stores:
            s.wait()

    return gather_kernel


def kernel(x, atom_emb):
    batch = x.shape[0]
    vocab, d = atom_emb.shape
    gather_kernel = _build_gather(batch, vocab, d)
    return gather_kernel(x.astype(jnp.int32), atom_emb)


# speedup vs baseline: 1.0060x; 1.0060x over previous
import functools

import jax
import jax.numpy as jnp
from jax import lax
from jax.experimental import pallas as pl
from jax.experimental.pallas import tpu as pltpu
from jax.experimental.pallas import tpu_sc as plsc

IDX_CHUNK = 64


def _build_gather(batch: int, vocab: int, d: int):
    info = plsc.get_sparse_core_info()
    nw = info.num_cores * info.num_subcores
    b_per_w = batch // nw
    n_chunks = b_per_w // IDX_CHUNK
    mesh = plsc.VectorSubcoreMesh(core_axis_name="c", subcore_axis_name="s")

    @functools.partial(
        pl.kernel,
        mesh=mesh,
        out_type=jax.ShapeDtypeStruct((batch, d), jnp.float32),
        scratch_types=[
            pltpu.VMEM((b_per_w,), jnp.int32),
            pltpu.VMEM((b_per_w, d), jnp.float32),
            pltpu.VMEM_SHARED((vocab, d), jnp.float32),
        ]
        + [pltpu.SemaphoreType.DMA] * (n_chunks + 1),
    )
    def gather_kernel(idx_hbm, table_hbm, out_hbm, idx_v, rows_v, table_sh, *sems):
        gsems, ssem = sems[:n_chunks], sems[n_chunks]
        cid = lax.axis_index("c")
        sid = lax.axis_index("s")
        wid = sid * info.num_cores + cid
        base = wid * b_per_w

        @pl.when(sid == 0)
        def _():
            pltpu.sync_copy(table_hbm, table_sh)

        pltpu.sync_copy(idx_hbm.at[pl.ds(base, b_per_w)], idx_v)
        plsc.subcore_barrier()

        copies = []
        for j in range(n_chunks):
            copies.append(
                pltpu.async_copy(
                    table_sh.at[idx_v.at[pl.ds(j * IDX_CHUNK, IDX_CHUNK)]],
                    rows_v.at[pl.ds(j * IDX_CHUNK, IDX_CHUNK)],
                    gsems[j],
                )
            )
        stores = []
        for j in range(n_chunks):
            copies[j].wait()
            stores.append(
                pltpu.async_copy(
                    rows_v.at[pl.ds(j * IDX_CHUNK, IDX_CHUNK)],
                    out_hbm.at[pl.ds(base + j * IDX_CHUNK, IDX_CHUNK)],
                    ssem,
                )
            )
        for s in stores:
            s.wait()

    return gather_kernel


def kernel(x, atom_emb):
    batch = x.shape[0]
    vocab, d = atom_emb.shape
    gather_kernel = _build_gather(batch, vocab, d)
    return gather_kernel(x.astype(jnp.int32), atom_emb)
